# Initial kernel scaffold; baseline (speedup 1.0000x reference)
import jax
import jax.numpy as jnp
from jax.experimental import pallas as pl


def kernel(feature, edge_index, edge_weight, self_weight, W, b):
    raise NotImplementedError("write your pallas kernel here")


# speedup vs baseline: 5.8786x; 5.8786x over previous
import functools

import jax
import jax.numpy as jnp
from jax import lax
from jax.experimental import pallas as pl
from jax.experimental.pallas import tpu as pltpu
from jax.experimental.pallas import tpu_sc as plsc

N_NODES = 10000
D = 128
NC = 2
NS = 16
NW = NC * NS
G = 80
SB = 25
N_PAD = 10240
R_PER_SUB = N_PAD // NS


def _sc_agg(feature, e4, ew3, K):
    mesh = plsc.VectorSubcoreMesh(core_axis_name="c", subcore_axis_name="s")

    @functools.partial(
        pl.kernel,
        mesh=mesh,
        out_type=jax.ShapeDtypeStruct((NC, N_PAD, D), jnp.float32),
        scratch_types=[
            pltpu.VMEM((SB, 2, G), jnp.int32),
            pltpu.VMEM((SB * G,), jnp.float32),
            pltpu.VMEM((G, D), jnp.float32),
            pltpu.VMEM_SHARED((N_PAD, D), jnp.float32),
            pltpu.SemaphoreType.DMA,
        ],
    )
    def sc_agg(feat_hbm, e_hbm, ew_hbm, out_hbm, e_v, w_v, rows_v, agg_s,
               sem):
        cid = lax.axis_index("c")
        sid = lax.axis_index("s")
        wid = cid * NS + sid

        zeros = jnp.zeros((16,), jnp.float32)

        def zrow(r, carry):
            for j in range(D // 16):
                rows_v[r, pl.ds(j * 16, 16)] = zeros
            return carry

        lax.fori_loop(0, G, zrow, 0)
        for t in range(R_PER_SUB // G):
            pltpu.sync_copy(rows_v, agg_s.at[pl.ds(sid * R_PER_SUB + t * G, G)])
        plsc.subcore_barrier()

        def super_step(s, carry):
            pltpu.sync_copy(e_hbm.at[wid, pl.ds(s * SB, SB)], e_v)
            pltpu.sync_copy(ew_hbm.at[wid, s], w_v)

            def step(q, c1):
                pltpu.async_copy(feat_hbm.at[e_v.at[q, 0]], rows_v, sem).wait()

                def scale(u, c2):
                    w16 = w_v[pl.ds(q * G + u * 16, 16)] + 1.0
                    for t in range(16):
                        wb = lax.gather(
                            w16, jnp.full((16, 1), t, jnp.int32),
                            lax.GatherDimensionNumbers(
                                offset_dims=(), collapsed_slice_dims=(0,),
                                start_index_map=(0,)),
                            slice_sizes=(1,),
                            mode=lax.GatherScatterMode.PROMISE_IN_BOUNDS)
                        i = u * 16 + t
                        for j in range(D // 16):
                            sl = pl.ds(j * 16, 16)
                            rows_v[i, sl] = rows_v[i, sl] * wb
                    return c2

                lax.fori_loop(0, G // 16, scale, 0)

                pltpu.sync_copy(rows_v, agg_s.at[e_v.at[q, 1]], add=True)
                return c1

            lax.fori_loop(0, SB, step, 0)
            return carry

        lax.fori_loop(0, K // SB, super_step, 0)
        plsc.subcore_barrier()

        for t in range(R_PER_SUB // G):
            r0 = sid * R_PER_SUB + t * G
            pltpu.sync_copy(agg_s.at[pl.ds(r0, G)], rows_v)
            pltpu.sync_copy(rows_v, out_hbm.at[cid, pl.ds(r0, G)])

    return sc_agg(feature, e4, ew3)


def _tc_body(f_ref, sw_ref, a0_ref, a1_ref, w_ref, b_ref, o_ref):
    h = f_ref[...] * (sw_ref[...] + 1.0) + a0_ref[...] + a1_ref[...]
    o_ref[...] = lax.dot_general(
        h, w_ref[...], (((1,), (1,)), ((), ())),
        preferred_element_type=jnp.float32,
    ) + b_ref[...]


def _tc_linear(feature, self_weight, agg0, agg1, W, b2):
    B = 1000
    grid = (N_NODES // B,)
    return pl.pallas_call(
        _tc_body,
        grid=grid,
        in_specs=[
            pl.BlockSpec((B, D), lambda i: (i, 0)),
            pl.BlockSpec((B, 1), lambda i: (i, 0)),
            pl.BlockSpec((B, D), lambda i: (i, 0)),
            pl.BlockSpec((B, D), lambda i: (i, 0)),
            pl.BlockSpec((D, D), lambda i: (0, 0)),
            pl.BlockSpec((1, D), lambda i: (0, 0)),
        ],
        out_specs=pl.BlockSpec((B, D), lambda i: (i, 0)),
        out_shape=jax.ShapeDtypeStruct((N_NODES, D), jnp.float32),
    )(feature, self_weight, agg0, agg1, W, b2)


def kernel(feature, edge_index, edge_weight, self_weight, W, b):
    E = edge_index.shape[1]
    K = E // (NW * G)
    src3 = edge_index[0].astype(jnp.int32).reshape(NW, K, G)
    dst3 = edge_index[1].astype(jnp.int32).reshape(NW, K, G)
    ew3 = edge_weight.astype(jnp.float32).reshape(NW, K // SB, SB * G)
    e4 = jnp.stack([src3, dst3], axis=2)
    agg2 = _sc_agg(feature, e4, ew3, K)
    return _tc_linear(feature, self_weight, agg2[0, :N_NODES],
                      agg2[1, :N_NODES], W, b.reshape(1, D))
